# baseline (speedup 1.0000x reference)
import jax
import jax.numpy as jnp
from jax.experimental import pallas as pl
from jax.experimental.pallas import tpu as pltpu

D = 1024
DFF = 4096
E = 64
DEXP = 1024
FF_BLK = 1024
N_FF = DFF // FF_BLK
NEG_BIG = -3.0e38
SPLIT = 2
DCHUNK = DEXP // SPLIT
N_E = E * SPLIT


def _body(x_ref, Wfc1_ref, bfc1_ref, Wfc2_ref, bfc2_ref, Wg_ref, bg_ref,
          W1_ref, b1_ref, W2_ref, b2_ref, out_ref,
          acc_ref, tokens_ref, w_ref):
    i = pl.program_id(0)

    @pl.when(i < N_FF)
    def _ffn():
        h = jnp.dot(x_ref[...], Wfc1_ref[...],
                    preferred_element_type=jnp.float32)
        h = jnp.maximum(h + bfc1_ref[0], 0.0)
        contrib = jnp.dot(h, Wfc2_ref[...], preferred_element_type=jnp.float32)

        @pl.when(i == 0)
        def _():
            acc_ref[...] = contrib

        @pl.when(i > 0)
        def _():
            acc_ref[...] += contrib

        @pl.when(i == N_FF - 1)
        def _router():
            tokens = acc_ref[...] + bfc2_ref[...]
            tokens_ref[...] = tokens
            logits = jnp.dot(tokens, Wg_ref[...],
                             preferred_element_type=jnp.float32)
            logits = logits + bg_ref[...]
            iota = jax.lax.broadcasted_iota(jnp.int32, logits.shape, 1)
            m1 = jnp.max(logits, axis=1, keepdims=True)
            i1 = jnp.min(jnp.where(logits == m1, iota, E), axis=1,
                         keepdims=True)
            sel1 = iota == i1
            masked = jnp.where(sel1, NEG_BIG, logits)
            m2 = jnp.max(masked, axis=1, keepdims=True)
            i2 = jnp.min(jnp.where(masked == m2, iota, E), axis=1,
                         keepdims=True)
            sel2 = iota == i2
            e2 = jnp.exp(m2 - m1)
            denom = 1.0 + e2
            w = jnp.where(sel1, 1.0 / denom, 0.0) + \
                jnp.where(sel2, e2 / denom, 0.0)
            w_ref[...] = w
            acc_ref[...] = jnp.dot(w, b2_ref[...],
                                   preferred_element_type=jnp.float32)

    @pl.when(i >= N_FF)
    def _expert():
        e = (i - N_FF) // SPLIT
        h1 = jnp.dot(tokens_ref[...], W1_ref[0],
                     preferred_element_type=jnp.float32)
        h1 = jnp.maximum(h1 + b1_ref[0], 0.0)
        onehot = (jax.lax.broadcasted_iota(jnp.int32, (E, 1), 0) == e
                  ).astype(jnp.float32)
        wcol = jnp.dot(w_ref[...], onehot, preferred_element_type=jnp.float32)
        acc_ref[...] += jnp.dot(h1 * wcol, W2_ref[0],
                                preferred_element_type=jnp.float32)

        @pl.when(i == pl.num_programs(0) - 1)
        def _():
            out_ref[...] = acc_ref[...]


def kernel(x, W_fc1, b_fc1, W_fc2, b_fc2, Wg, bg, W1, b1, W2, b2):
    B, S, _ = x.shape
    T = B * S
    xt = x.reshape(T, D)

    def ffc(i):
        return jnp.minimum(i, N_FF - 1)

    def eix(i):
        k = jnp.clip(i - N_FF, 0, N_E - 1)
        return k // SPLIT, k % SPLIT

    out = pl.pallas_call(
        _body,
        grid=(N_FF + N_E,),
        in_specs=[
            pl.BlockSpec((T, D), lambda i: (0, 0)),
            pl.BlockSpec((D, FF_BLK), lambda i: (0, ffc(i))),
            pl.BlockSpec((1, 1, FF_BLK), lambda i: (ffc(i), 0, 0)),
            pl.BlockSpec((FF_BLK, D), lambda i: (ffc(i), 0)),
            pl.BlockSpec((1, D), lambda i: (0, 0)),
            pl.BlockSpec((D, E), lambda i: (0, 0)),
            pl.BlockSpec((1, E), lambda i: (0, 0)),
            pl.BlockSpec((1, D, DCHUNK),
                         lambda i: (eix(i)[0], 0, eix(i)[1])),
            pl.BlockSpec((1, 1, DCHUNK),
                         lambda i: (eix(i)[0], 0, eix(i)[1])),
            pl.BlockSpec((1, DCHUNK, D),
                         lambda i: (eix(i)[0], eix(i)[1], 0)),
            pl.BlockSpec((E, D), lambda i: (0, 0)),
        ],
        out_specs=pl.BlockSpec((T, D), lambda i: (0, 0)),
        out_shape=jax.ShapeDtypeStruct((T, D), jnp.float32),
        scratch_shapes=[
            pltpu.VMEM((T, D), jnp.float32),
            pltpu.VMEM((T, D), jnp.float32),
            pltpu.VMEM((T, E), jnp.float32),
        ],
    )(xt, W_fc1, b_fc1.reshape(N_FF, 1, FF_BLK), W_fc2,
      b_fc2.reshape(1, D), Wg, bg.reshape(1, E),
      W1, b1.reshape(E, 1, DEXP), W2, b2)

    return (out.reshape(B, S, D),)


# speedup vs baseline: 1.2663x; 1.2663x over previous
import jax
import jax.numpy as jnp
from jax.experimental import pallas as pl
from jax.experimental.pallas import tpu as pltpu

D = 1024
DFF = 4096
E = 64
DEXP = 1024
FF_BLK = 1024
N_FF = DFF // FF_BLK
NEG_BIG = -3.0e38
SPLIT = 2
DCHUNK = DEXP // SPLIT
N_E = E * SPLIT


def _body(x_ref, Wfc1_ref, bfc1_ref, Wfc2_ref, bfc2_ref, Wg_ref, bg_ref,
          W1a_ref, W1b_ref, b1a_ref, b1b_ref, W2a_ref, W2b_ref, b2_ref,
          out_ref, acc_ref, tokens_ref, w_ref):
    i = pl.program_id(0)

    @pl.when(i < N_FF)
    def _ffn():
        h = jnp.dot(x_ref[...], Wfc1_ref[...],
                    preferred_element_type=jnp.float32)
        h = jnp.maximum(h + bfc1_ref[0], 0.0)
        contrib = jnp.dot(h, Wfc2_ref[...], preferred_element_type=jnp.float32)

        @pl.when(i == 0)
        def _():
            acc_ref[...] = contrib

        @pl.when(i > 0)
        def _():
            acc_ref[...] += contrib

        @pl.when(i == N_FF - 1)
        def _router():
            tokens = acc_ref[...] + bfc2_ref[...]
            tokens_ref[...] = tokens
            logits = jnp.dot(tokens, Wg_ref[...],
                             preferred_element_type=jnp.float32)
            logits = logits + bg_ref[...]
            iota = jax.lax.broadcasted_iota(jnp.int32, logits.shape, 1)
            m1 = jnp.max(logits, axis=1, keepdims=True)
            i1 = jnp.min(jnp.where(logits == m1, iota, E), axis=1,
                         keepdims=True)
            sel1 = iota == i1
            masked = jnp.where(sel1, NEG_BIG, logits)
            m2 = jnp.max(masked, axis=1, keepdims=True)
            i2 = jnp.min(jnp.where(masked == m2, iota, E), axis=1,
                         keepdims=True)
            sel2 = iota == i2
            e2 = jnp.exp(m2 - m1)
            denom = 1.0 + e2
            w = jnp.where(sel1, 1.0 / denom, 0.0) + \
                jnp.where(sel2, e2 / denom, 0.0)
            w_ref[...] = w
            acc_ref[...] = jnp.dot(w, b2_ref[...],
                                   preferred_element_type=jnp.float32)

    @pl.when(i >= N_FF)
    def _expert():
        e = i - N_FF
        onehot = (jax.lax.broadcasted_iota(jnp.int32, (E, 1), 0) == e
                  ).astype(jnp.float32)
        wcol = jnp.dot(w_ref[...], onehot, preferred_element_type=jnp.float32)
        contrib = acc_ref[...]
        for W1h_ref, b1h_ref, W2h_ref in ((W1a_ref, b1a_ref, W2a_ref),
                                          (W1b_ref, b1b_ref, W2b_ref)):
            h1 = jnp.dot(tokens_ref[...], W1h_ref[0],
                         preferred_element_type=jnp.float32)
            h1 = jnp.maximum(h1 + b1h_ref[0], 0.0)
            contrib += jnp.dot(h1 * wcol, W2h_ref[0],
                               preferred_element_type=jnp.float32)
        acc_ref[...] = contrib

        @pl.when(i == pl.num_programs(0) - 1)
        def _():
            out_ref[...] = acc_ref[...]


def kernel(x, W_fc1, b_fc1, W_fc2, b_fc2, Wg, bg, W1, b1, W2, b2):
    B, S, _ = x.shape
    T = B * S
    xt = x.reshape(T, D)

    def ffc(i):
        return jnp.minimum(i, N_FF - 1)

    def eix(i):
        return jnp.clip(i - N_FF, 0, E - 1)

    out = pl.pallas_call(
        _body,
        grid=(N_FF + E,),
        in_specs=[
            pl.BlockSpec((T, D), lambda i: (0, 0)),
            pl.BlockSpec((D, FF_BLK), lambda i: (0, ffc(i))),
            pl.BlockSpec((1, 1, FF_BLK), lambda i: (ffc(i), 0, 0)),
            pl.BlockSpec((FF_BLK, D), lambda i: (ffc(i), 0)),
            pl.BlockSpec((1, D), lambda i: (0, 0)),
            pl.BlockSpec((D, E), lambda i: (0, 0)),
            pl.BlockSpec((1, E), lambda i: (0, 0)),
            pl.BlockSpec((1, D, DCHUNK), lambda i: (eix(i), 0, 0)),
            pl.BlockSpec((1, D, DCHUNK), lambda i: (eix(i), 0, 1)),
            pl.BlockSpec((1, 1, DCHUNK), lambda i: (eix(i), 0, 0)),
            pl.BlockSpec((1, 1, DCHUNK), lambda i: (eix(i), 0, 1)),
            pl.BlockSpec((1, DCHUNK, D), lambda i: (eix(i), 0, 0)),
            pl.BlockSpec((1, DCHUNK, D), lambda i: (eix(i), 1, 0)),
            pl.BlockSpec((E, D), lambda i: (0, 0)),
        ],
        out_specs=pl.BlockSpec((T, D), lambda i: (0, 0)),
        out_shape=jax.ShapeDtypeStruct((T, D), jnp.float32),
        scratch_shapes=[
            pltpu.VMEM((T, D), jnp.float32),
            pltpu.VMEM((T, D), jnp.float32),
            pltpu.VMEM((T, E), jnp.float32),
        ],
    )(xt, W_fc1, b_fc1.reshape(N_FF, 1, FF_BLK), W_fc2,
      b_fc2.reshape(1, D), Wg, bg.reshape(1, E),
      W1, W1, b1.reshape(E, 1, DEXP), b1.reshape(E, 1, DEXP), W2, W2, b2)

    return (out.reshape(B, S, D),)
